# Initial kernel scaffold; baseline (speedup 1.0000x reference)
import jax
import jax.numpy as jnp
from jax.experimental import pallas as pl


def kernel(x, edge_index, W1, b1, W2, b2):
    raise NotImplementedError("write your pallas kernel here")


# speedup vs baseline: 12.7393x; 12.7393x over previous
import functools

import jax
import jax.numpy as jnp
from jax import lax
from jax.experimental import pallas as pl
from jax.experimental.pallas import tpu as pltpu
from jax.experimental.pallas import tpu_sc as plsc

N_NODES = 10000
D_FEAT = 128
HIDDEN = 16
N_ACTIONS = 4
N_EDGES = 320000

NC = 2
NS = 16
NW = NC * NS
EPW = N_EDGES // NW
CHUNK = 80
NCHUNK = EPW // CHUNK
ROWS_PER_SUB = 624
ROWS_TAIL = N_NODES - NS * ROWS_PER_SUB

_HIGH = jax.lax.Precision.HIGHEST

_mesh = plsc.VectorSubcoreMesh(core_axis_name="c", subcore_axis_name="s")


@functools.partial(
    pl.kernel,
    out_type=jax.ShapeDtypeStruct((NC * N_NODES, HIDDEN), jnp.float32),
    mesh=_mesh,
    scratch_types=[
        pltpu.VMEM((CHUNK,), jnp.int32),
        pltpu.VMEM((CHUNK,), jnp.int32),
        pltpu.VMEM((CHUNK, HIDDEN), jnp.float32),
        pltpu.VMEM_SHARED((N_NODES, HIDDEN), jnp.float32),
        pltpu.SemaphoreType.DMA,
    ],
    compiler_params=pltpu.CompilerParams(use_tc_tiling_on_sc=False),
)
def _sc_segsum(vals_hbm, src_hbm, dst_hbm, zeros_hbm, out_hbm,
               src_v, dst_v, rows_v, acc_sh, sem):
    c = lax.axis_index("c")
    s = lax.axis_index("s")
    w = c * NS + s
    rbase = s * ROWS_PER_SUB
    tbase = NS * ROWS_PER_SUB
    pltpu.sync_copy(zeros_hbm.at[pl.ds(rbase, ROWS_PER_SUB)],
                    acc_sh.at[pl.ds(rbase, ROWS_PER_SUB)])

    @pl.when(s == 0)
    def _():
        pltpu.sync_copy(zeros_hbm.at[pl.ds(tbase, ROWS_TAIL)],
                        acc_sh.at[pl.ds(tbase, ROWS_TAIL)])

    plsc.subcore_barrier()
    ebase = w * EPW

    @pl.loop(0, NCHUNK)
    def _(i):
        off = ebase + i * CHUNK
        pltpu.sync_copy(src_hbm.at[pl.ds(off, CHUNK)], src_v)
        pltpu.async_copy(vals_hbm.at[src_v], rows_v, sem).wait()
        pltpu.sync_copy(dst_hbm.at[pl.ds(off, CHUNK)], dst_v)
        pltpu.sync_copy(rows_v, acc_sh.at[dst_v], add=True)

    plsc.subcore_barrier()
    pltpu.sync_copy(acc_sh.at[pl.ds(rbase, ROWS_PER_SUB)],
                    out_hbm.at[pl.ds(c * N_NODES + rbase, ROWS_PER_SUB)])

    @pl.when(s == 0)
    def _():
        pltpu.sync_copy(acc_sh.at[pl.ds(tbase, ROWS_TAIL)],
                        out_hbm.at[pl.ds(c * N_NODES + tbase, ROWS_TAIL)])


def _tc_matmul1(x, W1):
    def body(x_ref, w_ref, o_ref):
        o_ref[...] = jnp.dot(x_ref[...], w_ref[...], precision=_HIGH)
    return pl.pallas_call(
        body,
        out_shape=jax.ShapeDtypeStruct((N_NODES, HIDDEN), jnp.float32),
    )(x, W1)


def _tc_scale1(p0, p1, h1):
    def body(p0_ref, p1_ref, h1_ref, dis_ref, h1s_ref):
        deg = p0_ref[:, 0:1] + p1_ref[:, 0:1] + 1.0
        dis = jax.lax.rsqrt(deg)
        dis_ref[...] = dis
        h1s_ref[...] = dis * h1_ref[...]
    return pl.pallas_call(
        body,
        out_shape=(
            jax.ShapeDtypeStruct((N_NODES, 1), jnp.float32),
            jax.ShapeDtypeStruct((N_NODES, HIDDEN), jnp.float32),
        ),
    )(p0, p1, h1)


def _tc_mid(p0, p1, h1, dis, b1, W2p):
    def body(p0_ref, p1_ref, h1_ref, dis_ref, b1_ref, w2_ref,
             h2_ref, h2sp_ref):
        dis = dis_ref[...]
        s1 = p0_ref[...] + p1_ref[...]
        out1 = dis * s1 + dis * dis * h1_ref[...] + b1_ref[...]
        r = jnp.maximum(out1, 0.0)
        h2p = jnp.dot(r, w2_ref[...], precision=_HIGH)
        h2_ref[...] = h2p
        h2sp_ref[...] = dis * h2p
    return pl.pallas_call(
        body,
        out_shape=(
            jax.ShapeDtypeStruct((N_NODES, HIDDEN), jnp.float32),
            jax.ShapeDtypeStruct((N_NODES, HIDDEN), jnp.float32),
        ),
    )(p0, p1, h1, dis, b1, W2p)


def _tc_final(p0, p1, h2p, dis, b2):
    def body(p0_ref, p1_ref, h2_ref, dis_ref, b2_ref, o_ref):
        dis = dis_ref[...]
        s2 = (p0_ref[...] + p1_ref[...])[:, :N_ACTIONS]
        o_ref[...] = (dis * s2
                      + dis * dis * h2_ref[:, :N_ACTIONS]
                      + b2_ref[...])
    return pl.pallas_call(
        body,
        out_shape=jax.ShapeDtypeStruct((N_NODES, N_ACTIONS), jnp.float32),
    )(p0, p1, h2p, dis, b2)


def kernel(x, edge_index, W1, b1, W2, b2):
    src = edge_index[0]
    dst = edge_index[1]
    zeros = jnp.zeros((N_NODES, HIDDEN), jnp.float32)
    ones = jnp.ones((N_NODES, HIDDEN), jnp.float32)
    b1r = b1.reshape(1, HIDDEN)
    b2r = b2.reshape(1, N_ACTIONS)
    W2p = jnp.pad(W2, ((0, 0), (0, HIDDEN - N_ACTIONS)))

    degp = _sc_segsum(ones, src, dst, zeros)
    h1 = _tc_matmul1(x, W1)
    dis, h1s = _tc_scale1(degp[:N_NODES], degp[N_NODES:], h1)
    s1 = _sc_segsum(h1s, src, dst, zeros)
    h2p, h2sp = _tc_mid(s1[:N_NODES], s1[N_NODES:], h1, dis, b1r, W2p)
    s2 = _sc_segsum(h2sp, src, dst, zeros)
    return _tc_final(s2[:N_NODES], s2[N_NODES:], h2p, dis, b2r)


# speedup vs baseline: 43.3865x; 3.4057x over previous
import functools

import jax
import jax.numpy as jnp
from jax import lax
from jax.experimental import pallas as pl
from jax.experimental.pallas import tpu as pltpu
from jax.experimental.pallas import tpu_sc as plsc

N_NODES = 10000
D_FEAT = 128
HIDDEN = 16
N_ACTIONS = 4
N_EDGES = 320000

NC = 2
NS = 16
NW = NC * NS
EPW = N_EDGES // NW
CHUNK = 100
NCHUNK = EPW // CHUNK
NBUF = 4
DEG_FIRE = 10
ROWS_PER_SUB = 624
ROWS_TAIL = N_NODES - NS * ROWS_PER_SUB

_HIGH = jax.lax.Precision.HIGHEST

_mesh = plsc.VectorSubcoreMesh(core_axis_name="c", subcore_axis_name="s")
_sc_params = pltpu.CompilerParams(use_tc_tiling_on_sc=False)


def _acc_prologue(epacked_hbm, zeros_hbm, idx_v, acc_sh, isem, s, w):
    pltpu.async_copy(epacked_hbm.at[pl.ds(w * NCHUNK, NCHUNK)], idx_v, isem)
    rbase = s * ROWS_PER_SUB
    tbase = NS * ROWS_PER_SUB
    pltpu.sync_copy(zeros_hbm.at[pl.ds(rbase, ROWS_PER_SUB)],
                    acc_sh.at[pl.ds(rbase, ROWS_PER_SUB)])

    @pl.when(s == 0)
    def _():
        pltpu.sync_copy(zeros_hbm.at[pl.ds(tbase, ROWS_TAIL)],
                        acc_sh.at[pl.ds(tbase, ROWS_TAIL)])

    pltpu.make_async_copy(epacked_hbm.at[pl.ds(w * NCHUNK, NCHUNK)], idx_v,
                          isem).wait()
    plsc.subcore_barrier()


def _acc_epilogue(out_hbm, acc_sh, c, s):
    plsc.subcore_barrier()
    rbase = s * ROWS_PER_SUB
    tbase = NS * ROWS_PER_SUB
    pltpu.sync_copy(acc_sh.at[pl.ds(rbase, ROWS_PER_SUB)],
                    out_hbm.at[pl.ds(c * N_NODES + rbase, ROWS_PER_SUB)])

    @pl.when(s == 0)
    def _():
        pltpu.sync_copy(acc_sh.at[pl.ds(tbase, ROWS_TAIL)],
                        out_hbm.at[pl.ds(c * N_NODES + tbase, ROWS_TAIL)])


@functools.partial(
    pl.kernel,
    out_type=jax.ShapeDtypeStruct((NC * N_NODES, HIDDEN), jnp.float32),
    mesh=_mesh,
    scratch_types=[
        pltpu.VMEM((NCHUNK, 2, CHUNK), jnp.int32),
        pltpu.VMEM((CHUNK, HIDDEN), jnp.float32),
        pltpu.VMEM((CHUNK, HIDDEN), jnp.float32),
        pltpu.VMEM((CHUNK, HIDDEN), jnp.float32),
        pltpu.VMEM((CHUNK, HIDDEN), jnp.float32),
        pltpu.VMEM_SHARED((N_NODES, HIDDEN), jnp.float32),
        pltpu.SemaphoreType.DMA,
        pltpu.SemaphoreType.DMA,
        pltpu.SemaphoreType.DMA,
        pltpu.SemaphoreType.DMA,
        pltpu.SemaphoreType.DMA,
        pltpu.SemaphoreType.DMA,
        pltpu.SemaphoreType.DMA,
        pltpu.SemaphoreType.DMA,
        pltpu.SemaphoreType.DMA,
    ],
    compiler_params=_sc_params,
)
def _sc_segsum(vals_hbm, epacked_hbm, zeros_hbm, out_hbm,
               idx_v, b0, b1, b2, b3, acc_sh,
               isem, g0, g1, g2, g3, s0, s1, s2, s3):
    c = lax.axis_index("c")
    s = lax.axis_index("s")
    w = c * NS + s
    _acc_prologue(epacked_hbm, zeros_hbm, idx_v, acc_sh, isem, s, w)

    bufs = (b0, b1, b2, b3)
    gsems = (g0, g1, g2, g3)
    ssems = (s0, s1, s2, s3)

    def g_desc(i, b):
        return pltpu.make_async_copy(vals_hbm.at[idx_v.at[i, 0]], bufs[b],
                                     gsems[b])

    def s_desc(i, b):
        return pltpu.make_async_copy(bufs[b], acc_sh.at[idx_v.at[i, 1]],
                                     ssems[b])

    for b in range(NBUF):
        g_desc(b, b).start()

    @pl.loop(0, NCHUNK, step=NBUF)
    def _(i):
        for b in range(NBUF):
            g_desc(i + b, b).wait()
            pltpu.async_copy(bufs[b], acc_sh.at[idx_v.at[i + b, 1]],
                             ssems[b], add=True)
        for b in range(NBUF):
            s_desc(i + b, b).wait()
            nj = i + b + NBUF

            @pl.when(nj < NCHUNK)
            def _():
                g_desc(nj, b).start()

    _acc_epilogue(out_hbm, acc_sh, c, s)


@functools.partial(
    pl.kernel,
    out_type=jax.ShapeDtypeStruct((NC * N_NODES, HIDDEN), jnp.float32),
    mesh=_mesh,
    scratch_types=[
        pltpu.VMEM((NCHUNK, 2, CHUNK), jnp.int32),
        pltpu.VMEM((CHUNK, HIDDEN), jnp.float32),
        pltpu.VMEM_SHARED((N_NODES, HIDDEN), jnp.float32),
        pltpu.SemaphoreType.DMA,
        pltpu.SemaphoreType.DMA,
    ],
    compiler_params=_sc_params,
)
def _sc_deg(ones_hbm, epacked_hbm, zeros_hbm, out_hbm,
            idx_v, ones_v, acc_sh, isem, dsem):
    c = lax.axis_index("c")
    s = lax.axis_index("s")
    w = c * NS + s
    _acc_prologue(epacked_hbm, zeros_hbm, idx_v, acc_sh, isem, s, w)
    pltpu.sync_copy(ones_hbm, ones_v)

    @pl.loop(0, NCHUNK, step=DEG_FIRE)
    def _(i):
        for b in range(DEG_FIRE):
            pltpu.async_copy(ones_v, acc_sh.at[idx_v.at[i + b, 1]], dsem,
                             add=True)
        for b in range(DEG_FIRE):
            pltpu.make_async_copy(ones_v, acc_sh.at[idx_v.at[i + b, 1]],
                                  dsem).wait()

    _acc_epilogue(out_hbm, acc_sh, c, s)


def _tc_matmul1(x, W1):
    def body(x_ref, w_ref, o_ref):
        o_ref[...] = jnp.dot(x_ref[...], w_ref[...], precision=_HIGH)
    return pl.pallas_call(
        body,
        out_shape=jax.ShapeDtypeStruct((N_NODES, HIDDEN), jnp.float32),
    )(x, W1)


def _tc_scale1(p0, p1, h1):
    def body(p0_ref, p1_ref, h1_ref, dis_ref, h1s_ref):
        deg = p0_ref[:, 0:1] + p1_ref[:, 0:1] + 1.0
        dis = jax.lax.rsqrt(deg)
        dis_ref[...] = dis
        h1s_ref[...] = dis * h1_ref[...]
    return pl.pallas_call(
        body,
        out_shape=(
            jax.ShapeDtypeStruct((N_NODES, 1), jnp.float32),
            jax.ShapeDtypeStruct((N_NODES, HIDDEN), jnp.float32),
        ),
    )(p0, p1, h1)


def _tc_mid(p0, p1, h1, dis, b1, W2p):
    def body(p0_ref, p1_ref, h1_ref, dis_ref, b1_ref, w2_ref,
             h2_ref, h2sp_ref):
        dis = dis_ref[...]
        s1 = p0_ref[...] + p1_ref[...]
        out1 = dis * s1 + dis * dis * h1_ref[...] + b1_ref[...]
        r = jnp.maximum(out1, 0.0)
        h2p = jnp.dot(r, w2_ref[...], precision=_HIGH)
        h2_ref[...] = h2p
        h2sp_ref[...] = dis * h2p
    return pl.pallas_call(
        body,
        out_shape=(
            jax.ShapeDtypeStruct((N_NODES, HIDDEN), jnp.float32),
            jax.ShapeDtypeStruct((N_NODES, HIDDEN), jnp.float32),
        ),
    )(p0, p1, h1, dis, b1, W2p)


def _tc_final(p0, p1, h2p, dis, b2):
    def body(p0_ref, p1_ref, h2_ref, dis_ref, b2_ref, o_ref):
        dis = dis_ref[...]
        s2 = (p0_ref[...] + p1_ref[...])[:, :N_ACTIONS]
        o_ref[...] = (dis * s2
                      + dis * dis * h2_ref[:, :N_ACTIONS]
                      + b2_ref[...])
    return pl.pallas_call(
        body,
        out_shape=jax.ShapeDtypeStruct((N_NODES, N_ACTIONS), jnp.float32),
    )(p0, p1, h2p, dis, b2)


def kernel(x, edge_index, W1, b1, W2, b2):
    src = edge_index[0]
    dst = edge_index[1]
    epacked = jnp.stack(
        [src.reshape(NW, NCHUNK, CHUNK), dst.reshape(NW, NCHUNK, CHUNK)],
        axis=2,
    ).reshape(NW * NCHUNK, 2, CHUNK)
    zeros = jnp.zeros((N_NODES, HIDDEN), jnp.float32)
    ones_blk = jnp.ones((CHUNK, HIDDEN), jnp.float32)
    b1r = b1.reshape(1, HIDDEN)
    b2r = b2.reshape(1, N_ACTIONS)
    W2p = jnp.pad(W2, ((0, 0), (0, HIDDEN - N_ACTIONS)))

    degp = _sc_deg(ones_blk, epacked, zeros)
    h1 = _tc_matmul1(x, W1)
    dis, h1s = _tc_scale1(degp[:N_NODES], degp[N_NODES:], h1)
    s1 = _sc_segsum(h1s, epacked, zeros)
    h2p, h2sp = _tc_mid(s1[:N_NODES], s1[N_NODES:], h1, dis, b1r, W2p)
    s2 = _sc_segsum(h2sp, epacked, zeros)
    return _tc_final(s2[:N_NODES], s2[N_NODES:], h2p, dis, b2r)


# speedup vs baseline: 53.7532x; 1.2389x over previous
import functools

import jax
import jax.numpy as jnp
from jax import lax
from jax.experimental import pallas as pl
from jax.experimental.pallas import tpu as pltpu
from jax.experimental.pallas import tpu_sc as plsc

N_NODES = 10000
D_FEAT = 128
HIDDEN = 16
N_ACTIONS = 4
N_EDGES = 320000

NC = 2
NS = 16
NW = NC * NS
EPW = N_EDGES // NW
CHUNK = 100
NCHUNK = EPW // CHUNK
NBUF = 4
DEG_FIRE = 10
ROWS_PER_SUB = 624
ROWS_TAIL = N_NODES - NS * ROWS_PER_SUB

_HIGH = jax.lax.Precision.HIGHEST

_mesh = plsc.VectorSubcoreMesh(core_axis_name="c", subcore_axis_name="s")
_sc_params = pltpu.CompilerParams(use_tc_tiling_on_sc=False)


def _zero_acc(zeros_hbm, acc_sh, s):
    rbase = s * ROWS_PER_SUB
    tbase = NS * ROWS_PER_SUB
    pltpu.sync_copy(zeros_hbm.at[pl.ds(rbase, ROWS_PER_SUB)],
                    acc_sh.at[pl.ds(rbase, ROWS_PER_SUB)])

    @pl.when(s == 0)
    def _():
        pltpu.sync_copy(zeros_hbm.at[pl.ds(tbase, ROWS_TAIL)],
                        acc_sh.at[pl.ds(tbase, ROWS_TAIL)])


def _acc_epilogue(out_hbm, acc_sh, c, s):
    plsc.subcore_barrier()
    rbase = s * ROWS_PER_SUB
    tbase = NS * ROWS_PER_SUB
    pltpu.sync_copy(acc_sh.at[pl.ds(rbase, ROWS_PER_SUB)],
                    out_hbm.at[pl.ds(c * N_NODES + rbase, ROWS_PER_SUB)])

    @pl.when(s == 0)
    def _():
        pltpu.sync_copy(acc_sh.at[pl.ds(tbase, ROWS_TAIL)],
                        out_hbm.at[pl.ds(c * N_NODES + tbase, ROWS_TAIL)])


def _make_segsum(width):
    @functools.partial(
        pl.kernel,
        out_type=jax.ShapeDtypeStruct((NC * N_NODES, width), jnp.float32),
        mesh=_mesh,
        scratch_types=[
            pltpu.VMEM((NCHUNK, CHUNK), jnp.int32),
            pltpu.VMEM((NCHUNK, CHUNK), jnp.int32),
            pltpu.VMEM((CHUNK, width), jnp.float32),
            pltpu.VMEM((CHUNK, width), jnp.float32),
            pltpu.VMEM((CHUNK, width), jnp.float32),
            pltpu.VMEM((CHUNK, width), jnp.float32),
            pltpu.VMEM_SHARED((N_NODES, width), jnp.float32),
            pltpu.SemaphoreType.DMA,
            pltpu.SemaphoreType.DMA,
            pltpu.SemaphoreType.DMA,
            pltpu.SemaphoreType.DMA,
            pltpu.SemaphoreType.DMA,
            pltpu.SemaphoreType.DMA,
            pltpu.SemaphoreType.DMA,
            pltpu.SemaphoreType.DMA,
            pltpu.SemaphoreType.DMA,
        ],
        compiler_params=_sc_params,
    )
    def seg(vals_hbm, eview_hbm, zeros_hbm, out_hbm,
            sidx, didx, b0, b1, b2, b3, acc_sh,
            isem, g0, g1, g2, g3, s0, s1, s2, s3):
        c = lax.axis_index("c")
        s = lax.axis_index("s")
        w = c * NS + s
        pltpu.async_copy(eview_hbm.at[0, w], sidx, isem)
        pltpu.async_copy(eview_hbm.at[1, w], didx, isem)
        _zero_acc(zeros_hbm, acc_sh, s)
        pltpu.make_async_copy(eview_hbm.at[0, w], sidx, isem).wait()
        pltpu.make_async_copy(eview_hbm.at[1, w], didx, isem).wait()
        plsc.subcore_barrier()

        bufs = (b0, b1, b2, b3)
        gsems = (g0, g1, g2, g3)
        ssems = (s0, s1, s2, s3)

        def g_desc(i, b):
            return pltpu.make_async_copy(vals_hbm.at[sidx.at[i]], bufs[b],
                                         gsems[b])

        def s_desc(i, b):
            return pltpu.make_async_copy(bufs[b], acc_sh.at[didx.at[i]],
                                         ssems[b])

        for b in range(NBUF):
            g_desc(b, b).start()

        @pl.loop(0, NCHUNK, step=NBUF)
        def _(i):
            for b in range(NBUF):
                g_desc(i + b, b).wait()
                pltpu.async_copy(bufs[b], acc_sh.at[didx.at[i + b]],
                                 ssems[b], add=True)
            for b in range(NBUF):
                s_desc(i + b, b).wait()
                nj = i + b + NBUF

                @pl.when(nj < NCHUNK)
                def _():
                    g_desc(nj, b).start()

        _acc_epilogue(out_hbm, acc_sh, c, s)

    return seg


_sc_seg16 = _make_segsum(HIDDEN)


@functools.partial(
    pl.kernel,
    out_type=jax.ShapeDtypeStruct((NC * N_NODES, HIDDEN), jnp.float32),
    mesh=_mesh,
    scratch_types=[
        pltpu.VMEM((NCHUNK, CHUNK), jnp.int32),
        pltpu.VMEM((CHUNK, HIDDEN), jnp.float32),
        pltpu.VMEM_SHARED((N_NODES, HIDDEN), jnp.float32),
        pltpu.SemaphoreType.DMA,
        pltpu.SemaphoreType.DMA,
    ],
    compiler_params=_sc_params,
)
def _sc_deg(ones_hbm, eview_hbm, zeros_hbm, out_hbm,
            didx, ones_v, acc_sh, isem, dsem):
    c = lax.axis_index("c")
    s = lax.axis_index("s")
    w = c * NS + s
    pltpu.async_copy(eview_hbm.at[1, w], didx, isem)
    _zero_acc(zeros_hbm, acc_sh, s)
    pltpu.sync_copy(ones_hbm, ones_v)
    pltpu.make_async_copy(eview_hbm.at[1, w], didx, isem).wait()
    plsc.subcore_barrier()

    @pl.loop(0, NCHUNK, step=DEG_FIRE)
    def _(i):
        for b in range(DEG_FIRE):
            pltpu.async_copy(ones_v, acc_sh.at[didx.at[i + b]], dsem,
                             add=True)
        for b in range(DEG_FIRE):
            pltpu.make_async_copy(ones_v, acc_sh.at[didx.at[i + b]],
                                  dsem).wait()

    _acc_epilogue(out_hbm, acc_sh, c, s)


def _tc_matmul1(x, W1):
    def body(x_ref, w_ref, o_ref):
        o_ref[...] = jnp.dot(x_ref[...], w_ref[...], precision=_HIGH)
    return pl.pallas_call(
        body,
        out_shape=jax.ShapeDtypeStruct((N_NODES, HIDDEN), jnp.float32),
    )(x, W1)


def _tc_scale1(degp, h1):
    def body(degp_ref, h1_ref, dis_ref, h1s_ref):
        dv = degp_ref[...]
        deg = dv[:N_NODES, 0:1] + dv[N_NODES:, 0:1] + 1.0
        dis = jax.lax.rsqrt(deg)
        dis_ref[...] = dis
        h1s_ref[...] = dis * h1_ref[...]
    return pl.pallas_call(
        body,
        out_shape=(
            jax.ShapeDtypeStruct((N_NODES, 1), jnp.float32),
            jax.ShapeDtypeStruct((N_NODES, HIDDEN), jnp.float32),
        ),
    )(degp, h1)


def _tc_mid(s1, h1, dis, b1, W2):
    def body(s1_ref, h1_ref, dis_ref, b1_ref, w2_ref, h2_ref, h2s_ref):
        dis = dis_ref[...]
        sv = s1_ref[...]
        out1 = (dis * (sv[:N_NODES] + sv[N_NODES:])
                + dis * dis * h1_ref[...] + b1_ref[...][None, :])
        r = jnp.maximum(out1, 0.0)
        w2p = jnp.concatenate(
            [w2_ref[...],
             jnp.zeros((HIDDEN, HIDDEN - N_ACTIONS), jnp.float32)], axis=1)
        h2p = jnp.dot(r, w2p, precision=_HIGH)
        h2_ref[...] = h2p[:, :N_ACTIONS]
        h2s_ref[...] = dis * h2p
    return pl.pallas_call(
        body,
        out_shape=(
            jax.ShapeDtypeStruct((N_NODES, N_ACTIONS), jnp.float32),
            jax.ShapeDtypeStruct((N_NODES, HIDDEN), jnp.float32),
        ),
    )(s1, h1, dis, b1, W2)


def _tc_final(s2, h2, dis, b2):
    def body(s2_ref, h2_ref, dis_ref, b2_ref, o_ref):
        dis = dis_ref[...]
        sv = (s2_ref[...])[:, :N_ACTIONS]
        o_ref[...] = (dis * (sv[:N_NODES] + sv[N_NODES:])
                      + dis * dis * h2_ref[...] + b2_ref[...][None, :])
    return pl.pallas_call(
        body,
        out_shape=jax.ShapeDtypeStruct((N_NODES, N_ACTIONS), jnp.float32),
    )(s2, h2, dis, b2)


def kernel(x, edge_index, W1, b1, W2, b2):
    eview = edge_index.reshape(2, NW, NCHUNK, CHUNK)
    zeros16 = jnp.zeros((N_NODES, HIDDEN), jnp.float32)
    ones_blk = jnp.ones((CHUNK, HIDDEN), jnp.float32)

    degp = _sc_deg(ones_blk, eview, zeros16)
    h1 = _tc_matmul1(x, W1)
    dis, h1s = _tc_scale1(degp, h1)
    s1 = _sc_seg16(h1s, eview, zeros16)
    h2, h2s = _tc_mid(s1, h1, dis, b1, W2)
    s2 = _sc_seg16(h2s, eview, zeros16)
    return _tc_final(s2, h2, dis, b2)


# speedup vs baseline: 59.0243x; 1.0981x over previous
import functools

import jax
import jax.numpy as jnp
from jax import lax
from jax.experimental import pallas as pl
from jax.experimental.pallas import tpu as pltpu
from jax.experimental.pallas import tpu_sc as plsc

N_NODES = 10000
D_FEAT = 128
HIDDEN = 16
N_ACTIONS = 4
N_EDGES = 320000

NC = 2
NS = 16
NW = NC * NS
EPW = N_EDGES // NW
CHUNK = 100
NCHUNK = EPW // CHUNK
NBUF = 10
DEG_FIRE = 10
ROWS_PER_SUB = 624
ROWS_TAIL = N_NODES - NS * ROWS_PER_SUB

_HIGH = jax.lax.Precision.HIGHEST

_mesh = plsc.VectorSubcoreMesh(core_axis_name="c", subcore_axis_name="s")
_sc_params = pltpu.CompilerParams(use_tc_tiling_on_sc=False)


def _zero_acc(zeros_hbm, acc_sh, s):
    rbase = s * ROWS_PER_SUB
    tbase = NS * ROWS_PER_SUB
    pltpu.sync_copy(zeros_hbm.at[pl.ds(rbase, ROWS_PER_SUB)],
                    acc_sh.at[pl.ds(rbase, ROWS_PER_SUB)])

    @pl.when(s == 0)
    def _():
        pltpu.sync_copy(zeros_hbm.at[pl.ds(tbase, ROWS_TAIL)],
                        acc_sh.at[pl.ds(tbase, ROWS_TAIL)])


def _acc_epilogue(out_hbm, acc_sh, c, s):
    plsc.subcore_barrier()
    rbase = s * ROWS_PER_SUB
    tbase = NS * ROWS_PER_SUB
    pltpu.sync_copy(acc_sh.at[pl.ds(rbase, ROWS_PER_SUB)],
                    out_hbm.at[pl.ds(c * N_NODES + rbase, ROWS_PER_SUB)])

    @pl.when(s == 0)
    def _():
        pltpu.sync_copy(acc_sh.at[pl.ds(tbase, ROWS_TAIL)],
                        out_hbm.at[pl.ds(c * N_NODES + tbase, ROWS_TAIL)])


def _make_segsum(width):
    @functools.partial(
        pl.kernel,
        out_type=jax.ShapeDtypeStruct((NC * N_NODES, width), jnp.float32),
        mesh=_mesh,
        scratch_types=(
            [pltpu.VMEM((NCHUNK, CHUNK), jnp.int32),
             pltpu.VMEM((NCHUNK, CHUNK), jnp.int32)]
            + [pltpu.VMEM((CHUNK, width), jnp.float32)] * NBUF
            + [pltpu.VMEM_SHARED((N_NODES, width), jnp.float32)]
            + [pltpu.SemaphoreType.DMA] * (1 + 2 * NBUF)
        ),
        compiler_params=_sc_params,
    )
    def seg(vals_hbm, eview_hbm, zeros_hbm, out_hbm, sidx, didx, *rest):
        bufs = rest[:NBUF]
        acc_sh = rest[NBUF]
        isem = rest[NBUF + 1]
        gsems = rest[NBUF + 2:NBUF + 2 + NBUF]
        ssems = rest[NBUF + 2 + NBUF:]
        c = lax.axis_index("c")
        s = lax.axis_index("s")
        w = c * NS + s
        pltpu.async_copy(eview_hbm.at[0, w], sidx, isem)
        pltpu.async_copy(eview_hbm.at[1, w], didx, isem)
        _zero_acc(zeros_hbm, acc_sh, s)
        pltpu.make_async_copy(eview_hbm.at[0, w], sidx, isem).wait()
        pltpu.make_async_copy(eview_hbm.at[1, w], didx, isem).wait()
        plsc.subcore_barrier()

        def g_desc(i, b):
            return pltpu.make_async_copy(vals_hbm.at[sidx.at[i]], bufs[b],
                                         gsems[b])

        def s_desc(i, b):
            return pltpu.make_async_copy(bufs[b], acc_sh.at[didx.at[i]],
                                         ssems[b])

        for b in range(NBUF):
            g_desc(b, b).start()

        @pl.loop(0, NCHUNK, step=NBUF)
        def _(i):
            for b in range(NBUF):
                g_desc(i + b, b).wait()
                pltpu.async_copy(bufs[b], acc_sh.at[didx.at[i + b]],
                                 ssems[b], add=True)
            for b in range(NBUF):
                s_desc(i + b, b).wait()
                nj = i + b + NBUF

                @pl.when(nj < NCHUNK)
                def _():
                    g_desc(nj, b).start()

        _acc_epilogue(out_hbm, acc_sh, c, s)

    return seg


_sc_seg16 = _make_segsum(HIDDEN)


@functools.partial(
    pl.kernel,
    out_type=jax.ShapeDtypeStruct((NC * N_NODES, HIDDEN), jnp.float32),
    mesh=_mesh,
    scratch_types=[
        pltpu.VMEM((NCHUNK, CHUNK), jnp.int32),
        pltpu.VMEM((CHUNK, HIDDEN), jnp.float32),
        pltpu.VMEM_SHARED((N_NODES, HIDDEN), jnp.float32),
        pltpu.SemaphoreType.DMA,
        pltpu.SemaphoreType.DMA,
    ],
    compiler_params=_sc_params,
)
def _sc_deg(ones_hbm, eview_hbm, zeros_hbm, out_hbm,
            didx, ones_v, acc_sh, isem, dsem):
    c = lax.axis_index("c")
    s = lax.axis_index("s")
    w = c * NS + s
    pltpu.async_copy(eview_hbm.at[1, w], didx, isem)
    _zero_acc(zeros_hbm, acc_sh, s)
    pltpu.sync_copy(ones_hbm, ones_v)
    pltpu.make_async_copy(eview_hbm.at[1, w], didx, isem).wait()
    plsc.subcore_barrier()

    @pl.loop(0, NCHUNK, step=DEG_FIRE)
    def _(i):
        for b in range(DEG_FIRE):
            pltpu.async_copy(ones_v, acc_sh.at[didx.at[i + b]], dsem,
                             add=True)
        for b in range(DEG_FIRE):
            pltpu.make_async_copy(ones_v, acc_sh.at[didx.at[i + b]],
                                  dsem).wait()

    _acc_epilogue(out_hbm, acc_sh, c, s)


def _tc_matmul1(x, W1):
    def body(x_ref, w_ref, o_ref):
        o_ref[...] = jnp.dot(x_ref[...], w_ref[...], precision=_HIGH)
    return pl.pallas_call(
        body,
        out_shape=jax.ShapeDtypeStruct((N_NODES, HIDDEN), jnp.float32),
    )(x, W1)


def _tc_scale1(degp, h1):
    def body(degp_ref, h1_ref, dis_ref, h1s_ref):
        dv = degp_ref[...]
        deg = dv[:N_NODES, 0:1] + dv[N_NODES:, 0:1] + 1.0
        dis = jax.lax.rsqrt(deg)
        dis_ref[...] = dis
        h1s_ref[...] = dis * h1_ref[...]
    return pl.pallas_call(
        body,
        out_shape=(
            jax.ShapeDtypeStruct((N_NODES, 1), jnp.float32),
            jax.ShapeDtypeStruct((N_NODES, HIDDEN), jnp.float32),
        ),
    )(degp, h1)


def _tc_mid(s1, h1, dis, b1, W2):
    def body(s1_ref, h1_ref, dis_ref, b1_ref, w2_ref, h2_ref, h2s_ref):
        dis = dis_ref[...]
        sv = s1_ref[...]
        out1 = (dis * (sv[:N_NODES] + sv[N_NODES:])
                + dis * dis * h1_ref[...] + b1_ref[...][None, :])
        r = jnp.maximum(out1, 0.0)
        w2p = jnp.concatenate(
            [w2_ref[...],
             jnp.zeros((HIDDEN, HIDDEN - N_ACTIONS), jnp.float32)], axis=1)
        h2p = jnp.dot(r, w2p, precision=_HIGH)
        h2_ref[...] = h2p[:, :N_ACTIONS]
        h2s_ref[...] = dis * h2p
    return pl.pallas_call(
        body,
        out_shape=(
            jax.ShapeDtypeStruct((N_NODES, N_ACTIONS), jnp.float32),
            jax.ShapeDtypeStruct((N_NODES, HIDDEN), jnp.float32),
        ),
    )(s1, h1, dis, b1, W2)


def _tc_final(s2, h2, dis, b2):
    def body(s2_ref, h2_ref, dis_ref, b2_ref, o_ref):
        dis = dis_ref[...]
        sv = (s2_ref[...])[:, :N_ACTIONS]
        o_ref[...] = (dis * (sv[:N_NODES] + sv[N_NODES:])
                      + dis * dis * h2_ref[...] + b2_ref[...][None, :])
    return pl.pallas_call(
        body,
        out_shape=jax.ShapeDtypeStruct((N_NODES, N_ACTIONS), jnp.float32),
    )(s2, h2, dis, b2)


def kernel(x, edge_index, W1, b1, W2, b2):
    eview = edge_index.reshape(2, NW, NCHUNK, CHUNK)
    zeros16 = jnp.zeros((N_NODES, HIDDEN), jnp.float32)
    ones_blk = jnp.ones((CHUNK, HIDDEN), jnp.float32)

    degp = _sc_deg(ones_blk, eview, zeros16)
    h1 = _tc_matmul1(x, W1)
    dis, h1s = _tc_scale1(degp, h1)
    s1 = _sc_seg16(h1s, eview, zeros16)
    h2, h2s = _tc_mid(s1, h1, dis, b1, W2)
    s2 = _sc_seg16(h2s, eview, zeros16)
    return _tc_final(s2, h2, dis, b2)


# speedup vs baseline: 61.6201x; 1.0440x over previous
import functools

import jax
import jax.numpy as jnp
from jax import lax
from jax.experimental import pallas as pl
from jax.experimental.pallas import tpu as pltpu
from jax.experimental.pallas import tpu_sc as plsc

N_NODES = 10000
D_FEAT = 128
HIDDEN = 16
N_ACTIONS = 4
N_EDGES = 320000

NC = 2
NS = 16
NW = NC * NS
EPW = N_EDGES // NW
CHUNK = 125
NCHUNK = EPW // CHUNK
NBUF = 10
DEG_FIRE = 10
ROWS_PER_SUB = 624
ROWS_TAIL = N_NODES - NS * ROWS_PER_SUB

_HIGH = jax.lax.Precision.HIGHEST

_mesh = plsc.VectorSubcoreMesh(core_axis_name="c", subcore_axis_name="s")
_sc_params = pltpu.CompilerParams(use_tc_tiling_on_sc=False)


def _zero_acc(zeros_hbm, acc_sh, s):
    rbase = s * ROWS_PER_SUB
    tbase = NS * ROWS_PER_SUB
    pltpu.sync_copy(zeros_hbm.at[pl.ds(rbase, ROWS_PER_SUB)],
                    acc_sh.at[pl.ds(rbase, ROWS_PER_SUB)])

    @pl.when(s == 0)
    def _():
        pltpu.sync_copy(zeros_hbm.at[pl.ds(tbase, ROWS_TAIL)],
                        acc_sh.at[pl.ds(tbase, ROWS_TAIL)])


def _acc_epilogue(out_hbm, acc_sh, c, s):
    plsc.subcore_barrier()
    rbase = s * ROWS_PER_SUB
    tbase = NS * ROWS_PER_SUB
    pltpu.sync_copy(acc_sh.at[pl.ds(rbase, ROWS_PER_SUB)],
                    out_hbm.at[pl.ds(c * N_NODES + rbase, ROWS_PER_SUB)])

    @pl.when(s == 0)
    def _():
        pltpu.sync_copy(acc_sh.at[pl.ds(tbase, ROWS_TAIL)],
                        out_hbm.at[pl.ds(c * N_NODES + tbase, ROWS_TAIL)])


def _make_segsum(width):
    @functools.partial(
        pl.kernel,
        out_type=jax.ShapeDtypeStruct((NC * N_NODES, width), jnp.float32),
        mesh=_mesh,
        scratch_types=(
            [pltpu.VMEM((NCHUNK, CHUNK), jnp.int32),
             pltpu.VMEM((NCHUNK, CHUNK), jnp.int32)]
            + [pltpu.VMEM((CHUNK, width), jnp.float32)] * NBUF
            + [pltpu.VMEM_SHARED((N_NODES, width), jnp.float32)]
            + [pltpu.SemaphoreType.DMA] * (1 + 2 * NBUF)
        ),
        compiler_params=_sc_params,
    )
    def seg(vals_hbm, eview_hbm, zeros_hbm, out_hbm, sidx, didx, *rest):
        bufs = rest[:NBUF]
        acc_sh = rest[NBUF]
        isem = rest[NBUF + 1]
        gsems = rest[NBUF + 2:NBUF + 2 + NBUF]
        ssems = rest[NBUF + 2 + NBUF:]
        c = lax.axis_index("c")
        s = lax.axis_index("s")
        w = c * NS + s
        pltpu.async_copy(eview_hbm.at[0, w], sidx, isem)
        pltpu.async_copy(eview_hbm.at[1, w], didx, isem)
        _zero_acc(zeros_hbm, acc_sh, s)
        pltpu.make_async_copy(eview_hbm.at[0, w], sidx, isem).wait()
        pltpu.make_async_copy(eview_hbm.at[1, w], didx, isem).wait()
        plsc.subcore_barrier()

        def g_desc(i, b):
            return pltpu.make_async_copy(vals_hbm.at[sidx.at[i]], bufs[b],
                                         gsems[b])

        def s_desc(i, b):
            return pltpu.make_async_copy(bufs[b], acc_sh.at[didx.at[i]],
                                         ssems[b])

        for b in range(NBUF):
            g_desc(b, b).start()

        @pl.loop(0, NCHUNK, step=NBUF)
        def _(i):
            for b in range(NBUF):
                g_desc(i + b, b).wait()
                pltpu.async_copy(bufs[b], acc_sh.at[didx.at[i + b]],
                                 ssems[b], add=True)
            for b in range(NBUF):
                s_desc(i + b, b).wait()
                nj = i + b + NBUF

                @pl.when(nj < NCHUNK)
                def _():
                    g_desc(nj, b).start()

        _acc_epilogue(out_hbm, acc_sh, c, s)

    return seg


_sc_seg16 = _make_segsum(HIDDEN)


@functools.partial(
    pl.kernel,
    out_type=jax.ShapeDtypeStruct((NC * N_NODES, HIDDEN), jnp.float32),
    mesh=_mesh,
    scratch_types=[
        pltpu.VMEM((NCHUNK, CHUNK), jnp.int32),
        pltpu.VMEM((CHUNK, HIDDEN), jnp.float32),
        pltpu.VMEM_SHARED((N_NODES, HIDDEN), jnp.float32),
        pltpu.SemaphoreType.DMA,
        pltpu.SemaphoreType.DMA,
    ],
    compiler_params=_sc_params,
)
def _sc_deg(ones_hbm, eview_hbm, zeros_hbm, out_hbm,
            didx, ones_v, acc_sh, isem, dsem):
    c = lax.axis_index("c")
    s = lax.axis_index("s")
    w = c * NS + s
    pltpu.async_copy(eview_hbm.at[1, w], didx, isem)
    _zero_acc(zeros_hbm, acc_sh, s)
    pltpu.sync_copy(ones_hbm, ones_v)
    pltpu.make_async_copy(eview_hbm.at[1, w], didx, isem).wait()
    plsc.subcore_barrier()

    @pl.loop(0, NCHUNK, step=DEG_FIRE)
    def _(i):
        for b in range(DEG_FIRE):
            pltpu.async_copy(ones_v, acc_sh.at[didx.at[i + b]], dsem,
                             add=True)
        for b in range(DEG_FIRE):
            pltpu.make_async_copy(ones_v, acc_sh.at[didx.at[i + b]],
                                  dsem).wait()

    _acc_epilogue(out_hbm, acc_sh, c, s)


def _tc_matmul1(x, W1):
    def body(x_ref, w_ref, o_ref):
        o_ref[...] = jnp.dot(x_ref[...], w_ref[...], precision=_HIGH)
    return pl.pallas_call(
        body,
        out_shape=jax.ShapeDtypeStruct((N_NODES, HIDDEN), jnp.float32),
    )(x, W1)


def _tc_scale1(degp, h1):
    def body(degp_ref, h1_ref, dis_ref, h1s_ref):
        dv = degp_ref[...]
        deg = dv[:N_NODES, 0:1] + dv[N_NODES:, 0:1] + 1.0
        dis = jax.lax.rsqrt(deg)
        dis_ref[...] = dis
        h1s_ref[...] = dis * h1_ref[...]
    return pl.pallas_call(
        body,
        out_shape=(
            jax.ShapeDtypeStruct((N_NODES, 1), jnp.float32),
            jax.ShapeDtypeStruct((N_NODES, HIDDEN), jnp.float32),
        ),
    )(degp, h1)


def _tc_mid(s1, h1, dis, b1, W2):
    def body(s1_ref, h1_ref, dis_ref, b1_ref, w2_ref, h2_ref, h2s_ref):
        dis = dis_ref[...]
        sv = s1_ref[...]
        out1 = (dis * (sv[:N_NODES] + sv[N_NODES:])
                + dis * dis * h1_ref[...] + b1_ref[...][None, :])
        r = jnp.maximum(out1, 0.0)
        w2p = jnp.concatenate(
            [w2_ref[...],
             jnp.zeros((HIDDEN, HIDDEN - N_ACTIONS), jnp.float32)], axis=1)
        h2p = jnp.dot(r, w2p, precision=_HIGH)
        h2_ref[...] = h2p[:, :N_ACTIONS]
        h2s_ref[...] = dis * h2p
    return pl.pallas_call(
        body,
        out_shape=(
            jax.ShapeDtypeStruct((N_NODES, N_ACTIONS), jnp.float32),
            jax.ShapeDtypeStruct((N_NODES, HIDDEN), jnp.float32),
        ),
    )(s1, h1, dis, b1, W2)


def _tc_final(s2, h2, dis, b2):
    def body(s2_ref, h2_ref, dis_ref, b2_ref, o_ref):
        dis = dis_ref[...]
        sv = (s2_ref[...])[:, :N_ACTIONS]
        o_ref[...] = (dis * (sv[:N_NODES] + sv[N_NODES:])
                      + dis * dis * h2_ref[...] + b2_ref[...][None, :])
    return pl.pallas_call(
        body,
        out_shape=jax.ShapeDtypeStruct((N_NODES, N_ACTIONS), jnp.float32),
    )(s2, h2, dis, b2)


def kernel(x, edge_index, W1, b1, W2, b2):
    eview = edge_index.reshape(2, NW, NCHUNK, CHUNK)
    zeros16 = jnp.zeros((N_NODES, HIDDEN), jnp.float32)
    ones_blk = jnp.ones((CHUNK, HIDDEN), jnp.float32)

    degp = _sc_deg(ones_blk, eview, zeros16)
    h1 = _tc_matmul1(x, W1)
    dis, h1s = _tc_scale1(degp, h1)
    s1 = _sc_seg16(h1s, eview, zeros16)
    h2, h2s = _tc_mid(s1, h1, dis, b1, W2)
    s2 = _sc_seg16(h2s, eview, zeros16)
    return _tc_final(s2, h2, dis, b2)


# speedup vs baseline: 96.9172x; 1.5728x over previous
import functools

import jax
import jax.numpy as jnp
from jax import lax
from jax.experimental import pallas as pl
from jax.experimental.pallas import tpu as pltpu
from jax.experimental.pallas import tpu_sc as plsc

N_NODES = 10000
D_FEAT = 128
HIDDEN = 16
N_ACTIONS = 4
N_EDGES = 320000

NC = 2
NS = 16
NW = NC * NS
EPW = N_EDGES // NW
CHUNK = 125
NCHUNK = EPW // CHUNK
NBUF = 10
DEG_FIRE = 10
ROWS_PER_SUB = 624
ROWS_TAIL = N_NODES - NS * ROWS_PER_SUB

PACK = 8
NP = N_NODES // PACK
FLAT = N_NODES * HIDDEN

_HIGH = jax.lax.Precision.HIGHEST

_mesh = plsc.VectorSubcoreMesh(core_axis_name="c", subcore_axis_name="s")
_sc_params = pltpu.CompilerParams(use_tc_tiling_on_sc=False)


def _zero_acc(zeros_hbm, acc_sh, s):
    rbase = s * ROWS_PER_SUB
    tbase = NS * ROWS_PER_SUB
    pltpu.sync_copy(zeros_hbm.at[pl.ds(rbase, ROWS_PER_SUB)],
                    acc_sh.at[pl.ds(rbase, ROWS_PER_SUB)])

    @pl.when(s == 0)
    def _():
        pltpu.sync_copy(zeros_hbm.at[pl.ds(tbase, ROWS_TAIL)],
                        acc_sh.at[pl.ds(tbase, ROWS_TAIL)])


def _acc_epilogue(out0_hbm, out1_hbm, acc_sh, c, s):
    plsc.subcore_barrier()
    rbase = s * ROWS_PER_SUB
    tbase = NS * ROWS_PER_SUB

    def _store(out_hbm):
        pltpu.sync_copy(acc_sh.at[pl.ds(rbase, ROWS_PER_SUB)],
                        out_hbm.at[pl.ds(rbase, ROWS_PER_SUB)])

        @pl.when(s == 0)
        def _():
            pltpu.sync_copy(acc_sh.at[pl.ds(tbase, ROWS_TAIL)],
                            out_hbm.at[pl.ds(tbase, ROWS_TAIL)])

    @pl.when(c == 0)
    def _():
        _store(out0_hbm)

    @pl.when(c == 1)
    def _():
        _store(out1_hbm)


def _make_segsum(width):
    @functools.partial(
        pl.kernel,
        out_type=[jax.ShapeDtypeStruct((N_NODES, width), jnp.float32),
                  jax.ShapeDtypeStruct((N_NODES, width), jnp.float32)],
        mesh=_mesh,
        scratch_types=(
            [pltpu.VMEM((NCHUNK, CHUNK), jnp.int32),
             pltpu.VMEM((NCHUNK, CHUNK), jnp.int32)]
            + [pltpu.VMEM((CHUNK, width), jnp.float32)] * NBUF
            + [pltpu.VMEM_SHARED((N_NODES, width), jnp.float32)]
            + [pltpu.SemaphoreType.DMA] * (1 + 2 * NBUF)
        ),
        compiler_params=_sc_params,
    )
    def seg(vals_hbm, eview_hbm, zeros_hbm, out0_hbm, out1_hbm,
            sidx, didx, *rest):
        bufs = rest[:NBUF]
        acc_sh = rest[NBUF]
        isem = rest[NBUF + 1]
        gsems = rest[NBUF + 2:NBUF + 2 + NBUF]
        ssems = rest[NBUF + 2 + NBUF:]
        c = lax.axis_index("c")
        s = lax.axis_index("s")
        w = c * NS + s
        pltpu.async_copy(eview_hbm.at[0, w], sidx, isem)
        pltpu.async_copy(eview_hbm.at[1, w], didx, isem)
        _zero_acc(zeros_hbm, acc_sh, s)
        pltpu.make_async_copy(eview_hbm.at[0, w], sidx, isem).wait()
        pltpu.make_async_copy(eview_hbm.at[1, w], didx, isem).wait()
        plsc.subcore_barrier()

        def g_desc(i, b):
            return pltpu.make_async_copy(vals_hbm.at[sidx.at[i]], bufs[b],
                                         gsems[b])

        def s_desc(i, b):
            return pltpu.make_async_copy(bufs[b], acc_sh.at[didx.at[i]],
                                         ssems[b])

        for b in range(NBUF):
            g_desc(b, b).start()

        @pl.loop(0, NCHUNK, step=NBUF)
        def _(i):
            for b in range(NBUF):
                g_desc(i + b, b).wait()
                pltpu.async_copy(bufs[b], acc_sh.at[didx.at[i + b]],
                                 ssems[b], add=True)
            for b in range(NBUF):
                s_desc(i + b, b).wait()
                nj = i + b + NBUF

                @pl.when(nj < NCHUNK)
                def _():
                    g_desc(nj, b).start()

        _acc_epilogue(out0_hbm, out1_hbm, acc_sh, c, s)

    return seg


_sc_seg16 = _make_segsum(HIDDEN)


@functools.partial(
    pl.kernel,
    out_type=[jax.ShapeDtypeStruct((N_NODES, HIDDEN), jnp.float32),
              jax.ShapeDtypeStruct((N_NODES, HIDDEN), jnp.float32)],
    mesh=_mesh,
    scratch_types=[
        pltpu.VMEM((NCHUNK, CHUNK), jnp.int32),
        pltpu.VMEM((CHUNK, HIDDEN), jnp.float32),
        pltpu.VMEM_SHARED((N_NODES, HIDDEN), jnp.float32),
        pltpu.SemaphoreType.DMA,
        pltpu.SemaphoreType.DMA,
    ],
    compiler_params=_sc_params,
)
def _sc_deg(ones_hbm, eview_hbm, zeros_hbm, out0_hbm, out1_hbm,
            didx, ones_v, acc_sh, isem, dsem):
    c = lax.axis_index("c")
    s = lax.axis_index("s")
    w = c * NS + s
    pltpu.async_copy(eview_hbm.at[1, w], didx, isem)
    _zero_acc(zeros_hbm, acc_sh, s)
    pltpu.sync_copy(ones_hbm, ones_v)
    pltpu.make_async_copy(eview_hbm.at[1, w], didx, isem).wait()
    plsc.subcore_barrier()

    @pl.loop(0, NCHUNK, step=DEG_FIRE)
    def _(i):
        for b in range(DEG_FIRE):
            pltpu.async_copy(ones_v, acc_sh.at[didx.at[i + b]], dsem,
                             add=True)
        for b in range(DEG_FIRE):
            pltpu.make_async_copy(ones_v, acc_sh.at[didx.at[i + b]],
                                  dsem).wait()

    _acc_epilogue(out0_hbm, out1_hbm, acc_sh, c, s)


def _tc_matmul1(xp, W1bd):
    def body(x_ref, w_ref, o_ref):
        o_ref[...] = jnp.dot(x_ref[...], w_ref[...], precision=_HIGH)
    return pl.pallas_call(
        body,
        out_shape=jax.ShapeDtypeStruct((NP, PACK * HIDDEN), jnp.float32),
    )(xp, W1bd)


def _tc_scale1(deg0, deg1, h1p):
    def body(d0_ref, d1_ref, h1_ref, dis_ref, h1s_ref):
        deg = (d0_ref[...].reshape(NP, PACK * HIDDEN)
               + d1_ref[...].reshape(NP, PACK * HIDDEN) + 1.0)
        dis = jax.lax.rsqrt(deg)
        dis_ref[...] = dis
        h1s_ref[...] = (dis * h1_ref[...]).reshape(FLAT)
    return pl.pallas_call(
        body,
        out_shape=(
            jax.ShapeDtypeStruct((NP, PACK * HIDDEN), jnp.float32),
            jax.ShapeDtypeStruct((FLAT,), jnp.float32),
        ),
    )(deg0, deg1, h1p)


def _tc_mid(s10, s11, h1p, disp, b1t, W2bd):
    def body(s0_ref, s1_ref, h1_ref, dis_ref, b1_ref, w2_ref, h2s_ref):
        dis = dis_ref[...]
        sv = (s0_ref[...].reshape(NP, PACK * HIDDEN)
              + s1_ref[...].reshape(NP, PACK * HIDDEN))
        out1 = dis * sv + dis * dis * h1_ref[...] + b1_ref[...][None, :]
        r = jnp.maximum(out1, 0.0)
        h2p = jnp.dot(r, w2_ref[...], precision=_HIGH)
        h2s_ref[...] = (dis * h2p).reshape(FLAT)
    return pl.pallas_call(
        body,
        out_shape=jax.ShapeDtypeStruct((FLAT,), jnp.float32),
    )(s10, s11, h1p, disp, b1t, W2bd)


def _tc_final(s20, s21, h2s, disp, b2t):
    def body(s0_ref, s1_ref, h2s_ref, dis_ref, b2_ref, o_ref):
        dis = dis_ref[...]
        sv = (s0_ref[...].reshape(NP, PACK * HIDDEN)
              + s1_ref[...].reshape(NP, PACK * HIDDEN))
        h2sp = h2s_ref[...].reshape(NP, PACK * HIDDEN)
        o_ref[...] = (dis * sv + dis * h2sp
                      + b2_ref[...][None, :]).reshape(FLAT)
    return pl.pallas_call(
        body,
        out_shape=jax.ShapeDtypeStruct((FLAT,), jnp.float32),
    )(s20, s21, h2s, disp, b2t)


def _blockdiag8(W):
    k, m = W.shape
    return (jnp.eye(PACK, dtype=W.dtype)[:, None, :, None]
            * W[None, :, None, :]).reshape(PACK * k, PACK * m)


def kernel(x, edge_index, W1, b1, W2, b2):
    eview = edge_index.reshape(2, NW, NCHUNK, CHUNK)
    zeros16 = jnp.zeros((N_NODES, HIDDEN), jnp.float32)
    ones_blk = jnp.ones((CHUNK, HIDDEN), jnp.float32)
    xp = x.reshape(NP, PACK * D_FEAT)
    W1bd = _blockdiag8(W1)
    W2p = jnp.pad(W2, ((0, 0), (0, HIDDEN - N_ACTIONS)))
    W2bd = _blockdiag8(W2p)
    b1t = jnp.tile(b1, PACK)
    b2t = jnp.tile(jnp.pad(b2, (0, HIDDEN - N_ACTIONS)), PACK)

    deg0, deg1 = _sc_deg(ones_blk, eview, zeros16)
    h1p = _tc_matmul1(xp, W1bd)
    disp, h1s = _tc_scale1(deg0.reshape(FLAT), deg1.reshape(FLAT), h1p)
    s10, s11 = _sc_seg16(h1s.reshape(N_NODES, HIDDEN), eview, zeros16)
    h2s = _tc_mid(s10.reshape(FLAT), s11.reshape(FLAT), h1p, disp, b1t, W2bd)
    s20, s21 = _sc_seg16(h2s.reshape(N_NODES, HIDDEN), eview, zeros16)
    out = _tc_final(s20.reshape(FLAT), s21.reshape(FLAT), h2s, disp, b2t)
    return out.reshape(N_NODES, HIDDEN)[:, :N_ACTIONS]
